# Initial kernel scaffold; baseline (speedup 1.0000x reference)
import jax
import jax.numpy as jnp
from jax.experimental import pallas as pl


def kernel(logits, labels):
    raise NotImplementedError("write your pallas kernel here")


# speedup vs baseline: 3.1542x; 3.1542x over previous
import math

import jax
import jax.numpy as jnp
from jax.experimental import pallas as pl

N_ROWS = 1024
N_COLS = 100000
NB = 8
RB = N_ROWS // NB
NV = 8
VB = 12544
S0_SCALE = math.sqrt(2.0) * math.log(N_COLS - 1)
BETA = 1.0


def _pass1_kernel(lab_ref, x_ref, s0_ref, s1_ref, t_ref):
    vb = pl.program_id(1)
    x = x_ref[...]
    cols = jax.lax.broadcasted_iota(jnp.int32, (RB, VB), 1) + vb * VB
    xm = jnp.where(cols < N_COLS, x, -100.0)
    lab = lab_ref[...].reshape(RB, 1)
    e0 = jnp.exp(S0_SCALE * xm)
    e1 = jnp.exp(xm)
    p0 = jnp.sum(e0, axis=1).reshape(1, RB, 1)
    p1 = jnp.sum(e1, axis=1).reshape(1, RB, 1)
    tp = jnp.max(jnp.where(cols == lab, xm, -2.0), axis=1).reshape(1, RB, 1)

    @pl.when(vb == 0)
    def _():
        s0_ref[...] = p0
        s1_ref[...] = p1
        t_ref[...] = tp

    @pl.when(vb != 0)
    def _():
        s0_ref[...] += p0
        s1_ref[...] += p1
        t_ref[...] = jnp.maximum(t_ref[...], tp)


def _kth_smallest(c, k, n_iter=48):
    def body(_, carry):
        lo, hi = carry
        mid = 0.5 * (lo + hi)
        cnt = jnp.sum((c <= mid).astype(jnp.float32))
        take_hi = cnt >= (k + 1)
        return (jnp.where(take_hi, lo, mid), jnp.where(take_hi, mid, hi))

    lo, hi = jax.lax.fori_loop(
        0, n_iter, body, (jnp.float32(-1.1), jnp.float32(1.1))
    )
    return hi


def _acos(x):
    ax = jnp.abs(x)
    p = jnp.float32(-0.0012624911)
    for c in (0.0066700901, -0.0170881256, 0.0308918810, -0.0501743046,
              0.0889789874, -0.2145988016, 1.5707963050):
        p = p * ax + jnp.float32(c)
    r = jnp.sqrt(jnp.maximum(0.0, 1.0 - ax)) * p
    return jnp.where(x >= 0.0, r, jnp.float32(math.pi) - r)


def _mid_kernel(s0_ref, s1_ref, t_ref, s_out, l2_out, mt_out):
    t = t_ref[...].reshape(NB, RB)
    S0 = s0_ref[...].reshape(NB, RB)
    S1 = s1_ref[...].reshape(NB, RB)
    sum0 = jnp.sum(S0) - jnp.sum(jnp.exp(S0_SCALE * t))
    b_avg = sum0 / N_ROWS
    c = jnp.clip(t, -1.0 + 1e-07, 1.0 - 1e-07)
    ca = _kth_smallest(c, N_ROWS // 2 - 1)
    cb = _kth_smallest(c, N_ROWS // 2)
    theta_med = 0.5 * (_acos(ca) + _acos(cb))
    cos_sum = ca * cb - jnp.sqrt(
        jnp.maximum(0.0, (1.0 - ca * ca)) * jnp.maximum(0.0, (1.0 - cb * cb))
    )
    cos_med = jnp.sqrt(jnp.maximum(0.0, 0.5 * (1.0 + cos_sum)))
    denom = jnp.where(
        theta_med < jnp.float32(math.pi / 4.0),
        cos_med,
        jnp.float32(math.cos(math.pi / 4.0)),
    )
    s = jnp.log(b_avg) / denom
    s_out[...] = jnp.reshape(s, (1, 1))
    l2_out[...] = jnp.reshape(jnp.mean(jnp.log(S1)) - jnp.mean(t), (1, 1))
    mt_out[...] = jnp.reshape(jnp.mean(t), (1, 1))


def _pass2_kernel(s_ref, x_ref, ss_ref):
    vb = pl.program_id(1)
    s = s_ref[...]
    x = x_ref[...]
    cols = jax.lax.broadcasted_iota(jnp.int32, (RB, VB), 1) + vb * VB
    e = jnp.where(cols < N_COLS, jnp.exp(s * x), 0.0)
    p = jnp.sum(e, axis=1).reshape(1, RB, 1)

    @pl.when(vb == 0)
    def _():
        ss_ref[...] = p

    @pl.when(vb != 0)
    def _():
        ss_ref[...] += p


def _final_kernel(ss_ref, t_ref, s_ref, l2_ref, out_ref):
    ss = ss_ref[...].reshape(NB, RB)
    t = t_ref[...].reshape(NB, RB)
    s = s_ref[...]
    loss1 = jnp.mean(jnp.log(ss)) - s * jnp.mean(t)
    out_ref[...] = (BETA * loss1 + l2_ref[...]) / (1.0 + BETA)


def kernel(logits, labels):
    labels3 = labels.astype(jnp.int32).reshape(NB, RB, 1)

    row_spec = pl.BlockSpec((1, RB, 1), lambda rb, vb: (rb, 0, 0))
    big_spec = pl.BlockSpec((RB, VB), lambda rb, vb: (rb, vb))
    rowstat = jax.ShapeDtypeStruct((NB, RB, 1), jnp.float32)
    scalar = jax.ShapeDtypeStruct((1, 1), jnp.float32)

    s0_rows, s1_rows, t_rows = pl.pallas_call(
        _pass1_kernel,
        grid=(NB, NV),
        in_specs=[row_spec, big_spec],
        out_specs=[row_spec, row_spec, row_spec],
        out_shape=[rowstat, rowstat, rowstat],
    )(labels3, logits)

    s_sc, l2_sc, mt_sc = pl.pallas_call(
        _mid_kernel,
        out_shape=[scalar, scalar, scalar],
    )(s0_rows, s1_rows, t_rows)

    del mt_sc

    scalar_spec = pl.BlockSpec((1, 1), lambda rb, vb: (0, 0))
    ss_rows = pl.pallas_call(
        _pass2_kernel,
        grid=(NB, NV),
        in_specs=[scalar_spec, big_spec],
        out_specs=row_spec,
        out_shape=rowstat,
    )(s_sc, logits)

    loss = pl.pallas_call(
        _final_kernel,
        out_shape=scalar,
    )(ss_rows, t_rows, s_sc, l2_sc)

    return loss[0, 0]
